# baseline (speedup 1.0000x reference)
import functools

import jax
import jax.numpy as jnp
from jax import lax
from jax.experimental import pallas as pl
from jax.experimental.pallas import tpu as pltpu
from jax.experimental.pallas import tpu_sc as plsc

_BATCH = 1024
_T = 50
_TA = 48
_TT = _T - _TA
_V = 1000
_D = 1000
_DP = 1024
_DA = 896
_NC = 2
_NS = 16
_NW = _NC * _NS
_BPW = _BATCH // _NW
_TROWS = _BPW * _TT
_TCH = 8
_NTCH = _TROWS // _TCH

_mesh = plsc.VectorSubcoreMesh(core_axis_name="c", subcore_axis_name="s")


@functools.partial(
    pl.kernel,
    out_type=(
        jax.ShapeDtypeStruct((_BATCH, _T, _D), jnp.float32),
        jax.ShapeDtypeStruct((_BATCH, _TA, _DP - _DA), jnp.float32),
        jax.ShapeDtypeStruct((_BATCH * _TT, _DP), jnp.float32),
    ),
    mesh=_mesh,
    scratch_types=[
        pltpu.VMEM((_BATCH * _TA // _NW,), jnp.int32),
        pltpu.VMEM((_TROWS,), jnp.int32),
        pltpu.VMEM((_TA, _DP), jnp.float32),
        pltpu.VMEM((_TA, _DP), jnp.float32),
        pltpu.VMEM((_TCH, _DP), jnp.float32),
        pltpu.VMEM((_TCH, _DP), jnp.float32),
        pltpu.SemaphoreType.DMA,
        pltpu.SemaphoreType.DMA,
        pltpu.SemaphoreType.DMA,
        pltpu.SemaphoreType.DMA,
    ],
)
def _gather(idxa_hbm, idxt_hbm, table_hbm, out_hbm, tail_hbm, trow_hbm,
            idxa_v, idxt_v, bufa0, bufa1, buft0, buft1, ga, sa, gt, st):
    cid = lax.axis_index("c")
    sid = lax.axis_index("s")
    wid = sid * _NC + cid
    base = wid * _BPW
    pltpu.sync_copy(idxa_hbm.at[pl.ds(base * _TA, _BPW * _TA)], idxa_v)
    pltpu.sync_copy(idxt_hbm.at[pl.ds(base * _TT, _TROWS)], idxt_v)

    bufsa = (bufa0, bufa1)
    bufst = (buft0, buft1)

    def start_gather(b, slot):
        pltpu.async_copy(
            table_hbm.at[idxa_v.at[pl.ds(b * _TA, _TA)]], bufsa[slot], ga
        )

    def wait_gather(b, slot):
        pltpu.make_async_copy(
            table_hbm.at[idxa_v.at[pl.ds(b * _TA, _TA)]], bufsa[slot], ga
        ).wait()

    def start_store(b, slot):
        pltpu.async_copy(
            bufsa[slot].at[:, pl.ds(0, _DA)],
            out_hbm.at[base + b, pl.ds(0, _TA), pl.ds(0, _DA)],
            sa,
        )
        pltpu.async_copy(
            bufsa[slot].at[:, pl.ds(_DA, _DP - _DA)],
            tail_hbm.at[base + b],
            sa,
        )

    def drain_store():
        pltpu.make_async_copy(table_hbm.at[pl.ds(0, _TA)], bufa0, sa).wait()

    start_gather(0, 0)
    npair = _BPW // 2

    def body(j, carry):
        b = 2 * j
        wait_gather(b, 0)

        @pl.when(j >= 1)
        def _():
            drain_store()

        start_store(b, 0)
        start_gather(b + 1, 1)
        wait_gather(b + 1, 1)
        drain_store()
        start_store(b + 1, 1)

        @pl.when(j + 1 < npair)
        def _():
            start_gather(b + 2, 0)

        return carry

    lax.fori_loop(0, npair, body, 0)
    drain_store()

    def t_gather(k, slot):
        return pltpu.async_copy(
            table_hbm.at[idxt_v.at[pl.ds(k * _TCH, _TCH)]], bufst[slot], gt
        )

    def t_store(k, slot):
        return pltpu.async_copy(
            bufst[slot], trow_hbm.at[pl.ds(wid * _TROWS + k * _TCH, _TCH)], st
        )

    t_gather(0, 0)
    tstores = [None, None]
    for k in range(_NTCH):
        slot = k % 2
        pltpu.make_async_copy(
            table_hbm.at[idxt_v.at[pl.ds(k * _TCH, _TCH)]], bufst[slot], gt
        ).wait()
        if tstores[1 - slot] is not None:
            tstores[1 - slot].wait()
            tstores[1 - slot] = None
        if k + 1 < _NTCH:
            t_gather(k + 1, 1 - slot)
        tstores[slot] = t_store(k, slot)
    for s in tstores:
        if s is not None:
            s.wait()


def kernel(input, token_embedding_table):
    idx = input.astype(jnp.int32)
    idxa = idx[:, :_TA].reshape(-1)
    idxt = idx[:, _TA:].reshape(-1)
    table_p = jnp.pad(token_embedding_table, ((0, 0), (0, _DP - _D)))
    main, tail, trow = _gather(idxa, idxt, table_p)
    out = lax.dynamic_update_slice(main, tail[:, :, : _D - _DA], (0, 0, _DA))
    trow_u = trow[:, :_D].reshape(_BATCH, _TT, _D)
    out = lax.dynamic_update_slice(out, trow_u, (0, _TA, 0))
    return out


# speedup vs baseline: 1.0120x; 1.0120x over previous
import functools

import jax
import jax.numpy as jnp
from jax import lax
from jax.experimental import pallas as pl
from jax.experimental.pallas import tpu as pltpu
from jax.experimental.pallas import tpu_sc as plsc

_BATCH = 1024
_T = 50
_TA = 48
_TT = _T - _TA
_V = 1000
_D = 1000
_DP = 1024
_DA = 896
_NC = 2
_NS = 16
_NW = _NC * _NS
_BPW = _BATCH // _NW
_CH = 24
_NCH = _BPW * 2
_TROWS = _BPW * _TT
_TCH = 8
_NTCH = _TROWS // _TCH

_mesh = plsc.VectorSubcoreMesh(core_axis_name="c", subcore_axis_name="s")


@functools.partial(
    pl.kernel,
    out_type=(
        jax.ShapeDtypeStruct((_BATCH, _T, _D), jnp.float32),
        jax.ShapeDtypeStruct((_BATCH, _TA, _DP - _DA), jnp.float32),
        jax.ShapeDtypeStruct((_BATCH * _TT, _DP), jnp.float32),
    ),
    mesh=_mesh,
    scratch_types=[
        pltpu.VMEM((_BATCH * _TA // _NW,), jnp.int32),
        pltpu.VMEM((_TROWS,), jnp.int32),
        pltpu.VMEM((_CH, _DP), jnp.float32),
        pltpu.VMEM((_CH, _DP), jnp.float32),
        pltpu.VMEM((_CH, _DP), jnp.float32),
        pltpu.VMEM((_CH, _DP), jnp.float32),
        pltpu.VMEM((_TCH, _DP), jnp.float32),
        pltpu.VMEM((_TCH, _DP), jnp.float32),
        pltpu.SemaphoreType.DMA,
        pltpu.SemaphoreType.DMA,
        pltpu.SemaphoreType.DMA,
        pltpu.SemaphoreType.DMA,
    ],
)
def _gather(idxa_hbm, idxt_hbm, table_hbm, out_hbm, tail_hbm, trow_hbm,
            idxa_v, idxt_v, ba0, ba1, ba2, ba3, buft0, buft1,
            ga, sa, gt, st):
    cid = lax.axis_index("c")
    sid = lax.axis_index("s")
    wid = sid * _NC + cid
    base = wid * _BPW
    pltpu.sync_copy(idxa_hbm.at[pl.ds(base * _TA, _BPW * _TA)], idxa_v)
    pltpu.sync_copy(idxt_hbm.at[pl.ds(base * _TT, _TROWS)], idxt_v)

    bufsa = (ba0, ba1, ba2, ba3)
    bufst = (buft0, buft1)

    def start_gather(c, slot):
        pltpu.async_copy(
            table_hbm.at[idxa_v.at[pl.ds(c * _CH, _CH)]], bufsa[slot], ga
        )

    def wait_gather(c, slot):
        pltpu.make_async_copy(
            table_hbm.at[idxa_v.at[pl.ds(c * _CH, _CH)]], bufsa[slot], ga
        ).wait()

    def start_store(c, slot, l):
        b = c // 2
        toff = (l % 2) * _CH
        pltpu.async_copy(
            bufsa[slot].at[:, pl.ds(0, _DA)],
            out_hbm.at[base + b, pl.ds(toff, _CH), pl.ds(0, _DA)],
            sa,
        )
        pltpu.async_copy(
            bufsa[slot].at[:, pl.ds(_DA, _DP - _DA)],
            tail_hbm.at[base + b, pl.ds(toff, _CH)],
            sa,
        )

    def drain_store():
        pltpu.make_async_copy(table_hbm.at[pl.ds(0, _CH)], ba0, sa).wait()

    start_gather(0, 0)
    start_gather(1, 1)
    ngroup = _NCH // 4

    def body(i, carry):
        c0 = 4 * i
        for l in range(4):
            c = c0 + l
            wait_gather(c, l)
            if l < 2:
                @pl.when(i >= 1)
                def _():
                    drain_store()
            else:
                drain_store()
            start_store(c, l, l)
            if l < 2:
                start_gather(c + 2, (l + 2) % 4)
            else:
                @pl.when(i + 1 < ngroup)
                def _():
                    start_gather(c + 2, (l + 2) % 4)
        return carry

    lax.fori_loop(0, ngroup, body, 0)
    drain_store()
    drain_store()

    def t_gather(k, slot):
        return pltpu.async_copy(
            table_hbm.at[idxt_v.at[pl.ds(k * _TCH, _TCH)]], bufst[slot], gt
        )

    def t_store(k, slot):
        return pltpu.async_copy(
            bufst[slot], trow_hbm.at[pl.ds(wid * _TROWS + k * _TCH, _TCH)], st
        )

    t_gather(0, 0)
    tstores = [None, None]
    for k in range(_NTCH):
        slot = k % 2
        pltpu.make_async_copy(
            table_hbm.at[idxt_v.at[pl.ds(k * _TCH, _TCH)]], bufst[slot], gt
        ).wait()
        if tstores[1 - slot] is not None:
            tstores[1 - slot].wait()
            tstores[1 - slot] = None
        if k + 1 < _NTCH:
            t_gather(k + 1, 1 - slot)
        tstores[slot] = t_store(k, slot)
    for s in tstores:
        if s is not None:
            s.wait()


def kernel(input, token_embedding_table):
    idx = input.astype(jnp.int32)
    idxa = idx[:, :_TA].reshape(-1)
    idxt = idx[:, _TA:].reshape(-1)
    table_p = jnp.pad(token_embedding_table, ((0, 0), (0, _DP - _D)))
    main, tail, trow = _gather(idxa, idxt, table_p)
    out = lax.dynamic_update_slice(main, tail[:, :, : _D - _DA], (0, 0, _DA))
    trow_u = trow[:, :_D].reshape(_BATCH, _TT, _D)
    out = lax.dynamic_update_slice(out, trow_u, (0, _TA, 0))
    return out
